# baseline (speedup 1.0000x reference)
import functools

import jax
import jax.numpy as jnp
from jax import lax
from jax.experimental import pallas as pl
from jax.experimental.pallas import tpu as pltpu
from jax.experimental.pallas import tpu_sc as plsc

_VOCAB = 1000000
_DIM = 32
_BATCH = 4096
_HIST = 200

_NC = 2
_NS = 16
_NW = _NC * _NS

_B = _BATCH * _HIST
_BPW = _B // _NW
_CHUNK = 256
_NCHUNKS = _BPW // _CHUNK
_NRING = 4
_LOOK = 2
_NGROUPS = _NCHUNKS // _NRING


def _build():
    mesh = plsc.VectorSubcoreMesh(core_axis_name="c", subcore_axis_name="s")

    @functools.partial(
        pl.kernel,
        mesh=mesh,
        out_type=jax.ShapeDtypeStruct((_NW, _BPW, _DIM), jnp.float32),
        scratch_types=[
            pltpu.VMEM((_NCHUNKS, _CHUNK), jnp.int32),
            pltpu.VMEM((_NRING, _CHUNK, _DIM), jnp.float32),
            pltpu.SemaphoreType.DMA((_NRING,)),
            pltpu.SemaphoreType.DMA((_NRING,)),
        ],
        compiler_params=pltpu.CompilerParams(use_tc_tiling_on_sc=False),
    )
    def gather_kernel(idx_hbm, table_hbm, out_hbm, idx_v, rows_v, gsem, osem):
        wid = lax.axis_index("s") * _NC + lax.axis_index("c")
        pltpu.sync_copy(idx_hbm.at[wid], idx_v)

        def start_gather(j, b):
            pltpu.async_copy(table_hbm.at[idx_v.at[j]], rows_v.at[b],
                             gsem.at[b])

        def wait_gather(j, b):
            pltpu.make_async_copy(table_hbm.at[idx_v.at[j]], rows_v.at[b],
                                  gsem.at[b]).wait()

        def start_write(j, b):
            pltpu.async_copy(rows_v.at[b],
                             out_hbm.at[wid, pl.ds(j * _CHUNK, _CHUNK)],
                             osem.at[b])

        def wait_write(j, b):
            pltpu.make_async_copy(rows_v.at[b],
                                  out_hbm.at[wid, pl.ds(j * _CHUNK, _CHUNK)],
                                  osem.at[b]).wait()

        for b in range(_LOOK):
            start_gather(b, b)

        def body(g, carry):
            for b in range(_NRING):
                j = g * _NRING + b
                wait_gather(j, b)
                start_write(j, b)
                jg = j + _LOOK
                bg = (b + _LOOK) % _NRING

                @pl.when(jg < _NCHUNKS)
                def _():
                    @pl.when(jg >= _NRING)
                    def _():
                        wait_write(jg - _NRING, bg)
                    start_gather(jg, bg)
            return carry

        lax.fori_loop(0, _NGROUPS, body, 0)

        for b in range(_NRING):
            j = _NCHUNKS - _NRING + b
            wait_write(j, b)

    return gather_kernel


_gather = _build()


def kernel(x, table):
    idx = x.reshape(_NW, _NCHUNKS, _CHUNK).astype(jnp.int32)
    out = _gather(idx, table)
    return out.reshape(_BATCH, _HIST, _DIM)


# speedup vs baseline: 1.0038x; 1.0038x over previous
import functools

import jax
import jax.numpy as jnp
from jax import lax
from jax.experimental import pallas as pl
from jax.experimental.pallas import tpu as pltpu
from jax.experimental.pallas import tpu_sc as plsc

_VOCAB = 1000000
_DIM = 32
_BATCH = 4096
_HIST = 200

_NC = 2
_NS = 16
_NW = _NC * _NS

_RPW = _BATCH // _NW
_NRING = 8
_LOOK = 4
_NGROUPS = _RPW // _NRING


def _build():
    mesh = plsc.VectorSubcoreMesh(core_axis_name="c", subcore_axis_name="s")

    @functools.partial(
        pl.kernel,
        mesh=mesh,
        out_type=jax.ShapeDtypeStruct((_BATCH, _HIST, _DIM), jnp.float32),
        scratch_types=[
            pltpu.VMEM((_RPW, _HIST), jnp.int32),
            pltpu.VMEM((_NRING, _HIST, _DIM), jnp.float32),
            pltpu.SemaphoreType.DMA((_NRING,)),
            pltpu.SemaphoreType.DMA((_NRING,)),
        ],
        compiler_params=pltpu.CompilerParams(use_tc_tiling_on_sc=False),
    )
    def gather_kernel(idx_hbm, table_hbm, out_hbm, idx_v, rows_v, gsem, osem):
        wid = lax.axis_index("s") * _NC + lax.axis_index("c")
        base = wid * _RPW
        pltpu.sync_copy(idx_hbm.at[pl.ds(base, _RPW), :], idx_v)

        def start_gather(j, b):
            pltpu.async_copy(table_hbm.at[idx_v.at[j]], rows_v.at[b],
                             gsem.at[b])

        def wait_gather(j, b):
            pltpu.make_async_copy(table_hbm.at[idx_v.at[j]], rows_v.at[b],
                                  gsem.at[b]).wait()

        def start_write(j, b):
            pltpu.async_copy(rows_v.at[b], out_hbm.at[base + j], osem.at[b])

        def wait_write(j, b):
            pltpu.make_async_copy(rows_v.at[b], out_hbm.at[base + j],
                                  osem.at[b]).wait()

        for b in range(_LOOK):
            start_gather(b, b)

        def body(g, carry):
            for b in range(_NRING):
                j = g * _NRING + b
                wait_gather(j, b)
                start_write(j, b)
                jg = j + _LOOK
                bg = (b + _LOOK) % _NRING

                @pl.when(jg < _RPW)
                def _():
                    @pl.when(jg >= _NRING)
                    def _():
                        wait_write(jg - _NRING, bg)
                    start_gather(jg, bg)
            return carry

        lax.fori_loop(0, _NGROUPS, body, 0)

        for b in range(_NRING):
            j = _RPW - _NRING + b
            wait_write(j, b)

    return gather_kernel


_gather = _build()


def kernel(x, table):
    return _gather(x.astype(jnp.int32), table)
